# baseline (speedup 1.0000x reference)
import numpy as np
import jax
import jax.numpy as jnp
from jax import lax
from jax.experimental import pallas as pl
from jax.experimental.pallas import tpu as pltpu

EPS = 0.01
W = 256
N = W * W
R, C = 512, 128
BS = 64

_INT_MIN = np.int32(-2147483648)


def _grid_xy_np():
    xs, ys = np.meshgrid(np.linspace(-1.0, 1.0, W), np.linspace(1.0, -1.0, W))
    xg = np.asarray(xs, np.float32).reshape(R, C)
    yg = np.asarray(ys, np.float32).reshape(R, C)
    return xg, yg


def _proj_body(xg_ref, yg_ref, d_ref, kinv_ref, rt_ref, k_ref,
               zneg_ref, skey_ref, pix_ref, mbit_ref):
    d = d_ref[0]
    x = xg_ref[...] * d
    y = yg_ref[...] * d
    zc = -d
    ones = jnp.ones_like(d)
    xys = jnp.concatenate([x, y, zc, ones], axis=0)

    t1 = jnp.dot(kinv_ref[0], xys, preferred_element_type=jnp.float32)
    t2 = jnp.dot(rt_ref[0], t1, preferred_element_type=jnp.float32)
    t3 = jnp.dot(k_ref[0], t2, preferred_element_type=jnp.float32)
    px = t3[0:1]
    py = t3[1:2]
    z = t3[2:3]
    zneg_ref[0] = -z

    maskz = jnp.abs(z) < EPS
    nz = -z
    sx = jnp.where(maskz, -10.0, px / nz)
    sy = jnp.where(maskz, -10.0, py / nz) * (-1.0)
    tx = (sx + 1.0) * 128.0
    ty = (sy + 1.0) * 128.0
    oob = (tx < 0.0) | (tx > 255.0) | (ty < 0.0) | (ty > 255.0)
    mbit_ref[0] = oob.astype(jnp.int32)
    xs_i = jnp.clip(tx.astype(jnp.int32), 0, 255)
    ys_i = jnp.clip(ty.astype(jnp.int32), 0, 255)
    pix_ref[0] = ys_i * 256 + xs_i

    ib = lax.bitcast_convert_type(nz, jnp.int32)
    skey_ref[0] = jnp.where(ib < 0, jnp.bitwise_xor(~ib, _INT_MIN), ib)


def _project(depth, K, K_inv, RT):
    xg, yg = _grid_xy_np()
    d = depth.reshape(BS, 1, N)
    zneg, skey, pix, mbit = pl.pallas_call(
        _proj_body,
        grid=(BS,),
        in_specs=[
            pl.BlockSpec((1, N), lambda b: (0, 0)),
            pl.BlockSpec((1, N), lambda b: (0, 0)),
            pl.BlockSpec((1, 1, N), lambda b: (b, 0, 0)),
            pl.BlockSpec((1, 4, 4), lambda b: (b, 0, 0)),
            pl.BlockSpec((1, 4, 4), lambda b: (b, 0, 0)),
            pl.BlockSpec((1, 4, 4), lambda b: (b, 0, 0)),
        ],
        out_specs=[pl.BlockSpec((1, 1, N), lambda b: (b, 0, 0))] * 4,
        out_shape=[
            jax.ShapeDtypeStruct((BS, 1, N), jnp.float32),
            jax.ShapeDtypeStruct((BS, 1, N), jnp.int32),
            jax.ShapeDtypeStruct((BS, 1, N), jnp.int32),
            jax.ShapeDtypeStruct((BS, 1, N), jnp.int32),
        ],
    )(jnp.asarray(xg).reshape(1, N), jnp.asarray(yg).reshape(1, N), d,
      K_inv, RT, K)
    return zneg, skey, pix, mbit


def _exch(x, d):
    if d >= C:
        dr = d // C
        up = pltpu.roll(x, R - dr, 0)
        dn = pltpu.roll(x, dr, 0)
        bit = (lax.broadcasted_iota(jnp.int32, (R, C), 0) & dr) != 0
    else:
        up = pltpu.roll(x, C - d, 1)
        dn = pltpu.roll(x, d, 1)
        bit = (lax.broadcasted_iota(jnp.int32, (R, C), 1) & d) != 0
    return jnp.where(bit, dn, up)


def _sort_body(skey_ref, pix_ref, mbit_ref, pixs_ref, w_ref):
    iota_e = (lax.broadcasted_iota(jnp.int32, (R, C), 0) * C
              + lax.broadcasted_iota(jnp.int32, (R, C), 1))
    key = skey_ref[0]
    pay = jnp.bitwise_or(jnp.left_shift(pix_ref[0], 16), iota_e)

    for k in range(1, 17):
        asc = ((iota_e >> k) & 1) == 0
        for d in [1 << s for s in range(k - 1, -1, -1)]:
            pkey = _exch(key, d)
            ppay = _exch(pay, d)
            idx_m = pay & 0xFFFF
            idx_p = ppay & 0xFFFF
            gt = (key > pkey) | ((key == pkey) & (idx_m > idx_p))
            bitd = (iota_e & d) != 0
            take = jnp.logical_xor(gt, bitd == asc)
            key = jnp.where(take, pkey, key)
            pay = jnp.where(take, ppay, pay)

    pixs_ref[0] = jnp.bitwise_and(lax.shift_right_logical(pay, 16), 0xFFFF)
    w_ref[0] = jnp.bitwise_or(jnp.left_shift(pay & 0xFFFF, 1), mbit_ref[0])


def _sort(skey, pix, mbit):
    pixs, w = pl.pallas_call(
        _sort_body,
        grid=(BS,),
        in_specs=[pl.BlockSpec((1, R, C), lambda b: (b, 0, 0))] * 3,
        out_specs=[pl.BlockSpec((1, R, C), lambda b: (b, 0, 0))] * 2,
        out_shape=[
            jax.ShapeDtypeStruct((BS, R, C), jnp.int32),
            jax.ShapeDtypeStruct((BS, R, C), jnp.int32),
        ],
    )(skey.reshape(BS, R, C), pix.reshape(BS, R, C), mbit.reshape(BS, R, C))
    return pixs.reshape(BS, N), w.reshape(BS, N)


def kernel(depth, K, K_inv, RTinv_cam1, RT_cam2):
    RT = jnp.matmul(RT_cam2, RTinv_cam1)
    zneg, skey, pix, mbit = _project(depth, K, K_inv, RT)

    pixs, wv = _sort(skey, pix, mbit)

    b_idx = jnp.broadcast_to(jnp.arange(BS)[:, None], pixs.shape)
    winner = jnp.full((BS, N), -1, dtype=jnp.int32)
    winner = winner.at[b_idx, pixs].set(wv)

    valid = winner >= 0
    i = lax.shift_right_logical(winner, 1) & 0xFFFF
    mb = (winner & 1).astype(jnp.float32) * 4.0
    xg, yg = _grid_xy_np()
    g0 = jnp.take(jnp.asarray(xg).reshape(N), i)
    g1 = jnp.take(jnp.asarray(yg).reshape(N), i)
    v0 = jnp.where(valid, g0 + mb, -2.0)
    v1 = jnp.where(valid, -g1 + mb, -2.0)
    bil = jnp.stack([v0, v1], axis=1).reshape(BS, 2, W, W)
    return bil, zneg.reshape(BS, 1, W, W)


# speedup vs baseline: 1.2231x; 1.2231x over previous
import numpy as np
import functools

import jax
import jax.numpy as jnp
from jax import lax
from jax.experimental import pallas as pl
from jax.experimental.pallas import tpu as pltpu
from jax.experimental.pallas import tpu_sc as plsc

EPS = 0.01
W = 256
N = W * W
R, C = 512, 128
BS = 64

_INT_MIN = np.int32(-2147483648)


def _grid_xy_np():
    xs, ys = np.meshgrid(np.linspace(-1.0, 1.0, W), np.linspace(1.0, -1.0, W))
    xg = np.asarray(xs, np.float32).reshape(R, C)
    yg = np.asarray(ys, np.float32).reshape(R, C)
    return xg, yg


def _proj_body(xg_ref, yg_ref, d_ref, kinv_ref, rt_ref, k_ref,
               zneg_ref, skey_ref, pix_ref, mbit_ref):
    d = d_ref[0]
    x = xg_ref[...] * d
    y = yg_ref[...] * d
    zc = -d
    ones = jnp.ones_like(d)
    xys = jnp.concatenate([x, y, zc, ones], axis=0)

    t1 = jnp.dot(kinv_ref[0], xys, preferred_element_type=jnp.float32)
    t2 = jnp.dot(rt_ref[0], t1, preferred_element_type=jnp.float32)
    t3 = jnp.dot(k_ref[0], t2, preferred_element_type=jnp.float32)
    px = t3[0:1]
    py = t3[1:2]
    z = t3[2:3]
    zneg_ref[0] = -z

    maskz = jnp.abs(z) < EPS
    nz = -z
    sx = jnp.where(maskz, -10.0, px / nz)
    sy = jnp.where(maskz, -10.0, py / nz) * (-1.0)
    tx = (sx + 1.0) * 128.0
    ty = (sy + 1.0) * 128.0
    oob = (tx < 0.0) | (tx > 255.0) | (ty < 0.0) | (ty > 255.0)
    mbit_ref[0] = oob.astype(jnp.int32)
    xs_i = jnp.clip(tx.astype(jnp.int32), 0, 255)
    ys_i = jnp.clip(ty.astype(jnp.int32), 0, 255)
    pix_ref[0] = ys_i * 256 + xs_i

    ib = lax.bitcast_convert_type(nz, jnp.int32)
    skey_ref[0] = jnp.where(ib < 0, jnp.bitwise_xor(~ib, _INT_MIN), ib)


def _project(depth, K, K_inv, RT):
    xg, yg = _grid_xy_np()
    d = depth.reshape(BS, 1, N)
    zneg, skey, pix, mbit = pl.pallas_call(
        _proj_body,
        grid=(BS,),
        in_specs=[
            pl.BlockSpec((1, N), lambda b: (0, 0)),
            pl.BlockSpec((1, N), lambda b: (0, 0)),
            pl.BlockSpec((1, 1, N), lambda b: (b, 0, 0)),
            pl.BlockSpec((1, 4, 4), lambda b: (b, 0, 0)),
            pl.BlockSpec((1, 4, 4), lambda b: (b, 0, 0)),
            pl.BlockSpec((1, 4, 4), lambda b: (b, 0, 0)),
        ],
        out_specs=[pl.BlockSpec((1, 1, N), lambda b: (b, 0, 0))] * 4,
        out_shape=[
            jax.ShapeDtypeStruct((BS, 1, N), jnp.float32),
            jax.ShapeDtypeStruct((BS, 1, N), jnp.int32),
            jax.ShapeDtypeStruct((BS, 1, N), jnp.int32),
            jax.ShapeDtypeStruct((BS, 1, N), jnp.int32),
        ],
    )(jnp.asarray(xg).reshape(1, N), jnp.asarray(yg).reshape(1, N), d,
      K_inv, RT, K)
    return zneg, skey, pix, mbit


def _exch(x, d):
    if d >= C:
        dr = d // C
        up = pltpu.roll(x, R - dr, 0)
        dn = pltpu.roll(x, dr, 0)
        bit = (lax.broadcasted_iota(jnp.int32, (R, C), 0) & dr) != 0
    else:
        up = pltpu.roll(x, C - d, 1)
        dn = pltpu.roll(x, d, 1)
        bit = (lax.broadcasted_iota(jnp.int32, (R, C), 1) & d) != 0
    return jnp.where(bit, dn, up)


def _sort_body(skey_ref, pix_ref, mbit_ref, pixs_ref, w_ref):
    iota_e = (lax.broadcasted_iota(jnp.int32, (R, C), 0) * C
              + lax.broadcasted_iota(jnp.int32, (R, C), 1))
    key = skey_ref[0]
    pay = jnp.bitwise_or(jnp.left_shift(pix_ref[0], 16), iota_e)

    for k in range(1, 17):
        asc = ((iota_e >> k) & 1) == 0
        for d in [1 << s for s in range(k - 1, -1, -1)]:
            pkey = _exch(key, d)
            ppay = _exch(pay, d)
            idx_m = pay & 0xFFFF
            idx_p = ppay & 0xFFFF
            gt = (key > pkey) | ((key == pkey) & (idx_m > idx_p))
            bitd = (iota_e & d) != 0
            take = jnp.logical_xor(gt, bitd == asc)
            key = jnp.where(take, pkey, key)
            pay = jnp.where(take, ppay, pay)

    pixs = jnp.bitwise_and(lax.shift_right_logical(pay, 16), 0xFFFF)
    cmod = lax.broadcasted_iota(jnp.int32, (R, C), 1) % 16
    dup_later = jnp.zeros((R, C), dtype=jnp.bool_)
    for o in range(1, 16):
        eq = pixs == pltpu.roll(pixs, C - o, 1)
        dup_later = dup_later | (eq & (cmod < 16 - o))
    valid = jnp.logical_not(dup_later).astype(jnp.int32)
    pixs_ref[0] = pixs
    w_ref[0] = jnp.bitwise_or(
        jnp.bitwise_or(jnp.left_shift(pay & 0xFFFF, 1), mbit_ref[0]),
        jnp.left_shift(valid, 17))


def _sort(skey, pix, mbit):
    pixs, w = pl.pallas_call(
        _sort_body,
        grid=(BS,),
        in_specs=[pl.BlockSpec((1, R, C), lambda b: (b, 0, 0))] * 3,
        out_specs=[pl.BlockSpec((1, R, C), lambda b: (b, 0, 0))] * 2,
        out_shape=[
            jax.ShapeDtypeStruct((BS, R, C), jnp.int32),
            jax.ShapeDtypeStruct((BS, R, C), jnp.int32),
        ],
    )(skey.reshape(BS, R, C), pix.reshape(BS, R, C), mbit.reshape(BS, R, C))
    return pixs.reshape(BS, N), w.reshape(BS, N)


_NC = 2
_NS = 16
_NW = _NC * _NS
_IMGS_PER_W = BS // _NW
_CH = 2048


def _scatter_tec_body(pixs_hbm, w_hbm, out_hbm, buf_v, pix_v, w_v):
    wid = lax.axis_index("s") * _NC + lax.axis_index("c")

    for li in range(_IMGS_PER_W):
        img = wid * _IMGS_PER_W + li
        base = img * N

        def init_body(i, carry):
            buf_v[pl.ds(i * 16, 16)] = jnp.full((16,), -1, jnp.int32)
            return carry

        lax.fori_loop(0, N // 16, init_body, 0)

        def chunk_body(c, carry):
            pltpu.sync_copy(pixs_hbm.at[pl.ds(base + c * _CH, _CH)], pix_v)
            pltpu.sync_copy(w_hbm.at[pl.ds(base + c * _CH, _CH)], w_v)

            def group_body(g, carry2):
                pixg = pix_v[pl.ds(g * 16, 16)]
                wg = w_v[pl.ds(g * 16, 16)]
                ok = jnp.bitwise_and(lax.shift_right_logical(wg, 17), 1) == 1
                plsc.store_scatter(buf_v, [pixg], wg, mask=ok)
                return carry2

            lax.fori_loop(0, _CH // 16, group_body, 0)
            return carry

        lax.fori_loop(0, N // _CH, chunk_body, 0)
        pltpu.sync_copy(buf_v, out_hbm.at[pl.ds(base, N)])


def _sc_scatter(pixs, wv):
    mesh = plsc.VectorSubcoreMesh(core_axis_name="c", subcore_axis_name="s")
    f = functools.partial(
        pl.kernel,
        mesh=mesh,
        compiler_params=pltpu.CompilerParams(needs_layout_passes=False),
        out_type=jax.ShapeDtypeStruct((BS * N,), jnp.int32),
        scratch_types=[
            pltpu.VMEM((N,), jnp.int32),
            pltpu.VMEM((_CH,), jnp.int32),
            pltpu.VMEM((_CH,), jnp.int32),
        ],
    )(_scatter_tec_body)
    return f(pixs.reshape(BS * N), wv.reshape(BS * N)).reshape(BS, N)


def kernel(depth, K, K_inv, RTinv_cam1, RT_cam2):
    RT = jnp.matmul(RT_cam2, RTinv_cam1)
    zneg, skey, pix, mbit = _project(depth, K, K_inv, RT)

    pixs, wv = _sort(skey, pix, mbit)
    winner = _sc_scatter(pixs, wv)

    valid = winner >= 0
    i = lax.shift_right_logical(winner, 1) & 0xFFFF
    mb = (winner & 1).astype(jnp.float32) * 4.0
    xg, yg = _grid_xy_np()
    g0 = jnp.take(jnp.asarray(xg).reshape(N), i)
    g1 = jnp.take(jnp.asarray(yg).reshape(N), i)
    v0 = jnp.where(valid, g0 + mb, -2.0)
    v1 = jnp.where(valid, -g1 + mb, -2.0)
    bil = jnp.stack([v0, v1], axis=1).reshape(BS, 2, W, W)
    return bil, zneg.reshape(BS, 1, W, W)


# speedup vs baseline: 1.2293x; 1.0051x over previous
import numpy as np
import functools

import jax
import jax.numpy as jnp
from jax import lax
from jax.experimental import pallas as pl
from jax.experimental.pallas import tpu as pltpu
from jax.experimental.pallas import tpu_sc as plsc

EPS = 0.01
W = 256
N = W * W
R, C = 512, 128
BS = 64

_INT_MIN = np.int32(-2147483648)


def _grid_xy_np():
    xs, ys = np.meshgrid(np.linspace(-1.0, 1.0, W), np.linspace(1.0, -1.0, W))
    xg = np.asarray(xs, np.float32).reshape(R, C)
    yg = np.asarray(ys, np.float32).reshape(R, C)
    return xg, yg


def _proj_body(xg_ref, yg_ref, d_ref, kinv_ref, rt_ref, k_ref,
               zneg_ref, skey_ref, pix_ref, mbit_ref):
    d = d_ref[0]
    x = xg_ref[...] * d
    y = yg_ref[...] * d
    zc = -d
    ones = jnp.ones_like(d)
    xys = jnp.concatenate([x, y, zc, ones], axis=0)

    t1 = jnp.dot(kinv_ref[0], xys, preferred_element_type=jnp.float32)
    t2 = jnp.dot(rt_ref[0], t1, preferred_element_type=jnp.float32)
    t3 = jnp.dot(k_ref[0], t2, preferred_element_type=jnp.float32)
    px = t3[0:1]
    py = t3[1:2]
    z = t3[2:3]
    zneg_ref[0] = -z

    maskz = jnp.abs(z) < EPS
    nz = -z
    sx = jnp.where(maskz, -10.0, px / nz)
    sy = jnp.where(maskz, -10.0, py / nz) * (-1.0)
    tx = (sx + 1.0) * 128.0
    ty = (sy + 1.0) * 128.0
    oob = (tx < 0.0) | (tx > 255.0) | (ty < 0.0) | (ty > 255.0)
    mbit_ref[0] = oob.astype(jnp.int32)
    xs_i = jnp.clip(tx.astype(jnp.int32), 0, 255)
    ys_i = jnp.clip(ty.astype(jnp.int32), 0, 255)
    pix_ref[0] = ys_i * 256 + xs_i

    ib = lax.bitcast_convert_type(nz, jnp.int32)
    skey_ref[0] = jnp.where(ib < 0, jnp.bitwise_xor(~ib, _INT_MIN), ib)


def _project(depth, K, K_inv, RT):
    xg, yg = _grid_xy_np()
    d = depth.reshape(BS, 1, N)
    zneg, skey, pix, mbit = pl.pallas_call(
        _proj_body,
        grid=(BS,),
        in_specs=[
            pl.BlockSpec((1, N), lambda b: (0, 0)),
            pl.BlockSpec((1, N), lambda b: (0, 0)),
            pl.BlockSpec((1, 1, N), lambda b: (b, 0, 0)),
            pl.BlockSpec((1, 4, 4), lambda b: (b, 0, 0)),
            pl.BlockSpec((1, 4, 4), lambda b: (b, 0, 0)),
            pl.BlockSpec((1, 4, 4), lambda b: (b, 0, 0)),
        ],
        out_specs=[pl.BlockSpec((1, 1, N), lambda b: (b, 0, 0))] * 4,
        out_shape=[
            jax.ShapeDtypeStruct((BS, 1, N), jnp.float32),
            jax.ShapeDtypeStruct((BS, 1, N), jnp.int32),
            jax.ShapeDtypeStruct((BS, 1, N), jnp.int32),
            jax.ShapeDtypeStruct((BS, 1, N), jnp.int32),
        ],
    )(jnp.asarray(xg).reshape(1, N), jnp.asarray(yg).reshape(1, N), d,
      K_inv, RT, K)
    return zneg, skey, pix, mbit


def _exch(x, d):
    if d < R:
        up = pltpu.roll(x, R - d, 0)
        dn = pltpu.roll(x, d, 0)
        bit = (lax.broadcasted_iota(jnp.int32, (R, C), 0) & d) != 0
    else:
        dl = d // R
        up = pltpu.roll(x, C - dl, 1)
        dn = pltpu.roll(x, dl, 1)
        bit = (lax.broadcasted_iota(jnp.int32, (R, C), 1) & dl) != 0
    return jnp.where(bit, dn, up)


def _sort_body(skey_ref, pix_ref, mbit_ref, pixs_ref, w_ref):
    i0 = lax.broadcasted_iota(jnp.int32, (R, C), 0)
    i1 = lax.broadcasted_iota(jnp.int32, (R, C), 1)
    iota_i = i0 * C + i1
    iota_e = i1 * R + i0
    key = skey_ref[0]
    pay = jnp.bitwise_or(jnp.left_shift(pix_ref[0], 16), iota_i)

    for k in range(1, 17):
        asc = ((iota_e >> k) & 1) == 0
        for d in [1 << s for s in range(k - 1, -1, -1)]:
            pkey = _exch(key, d)
            ppay = _exch(pay, d)
            idx_m = pay & 0xFFFF
            idx_p = ppay & 0xFFFF
            gt = (key > pkey) | ((key == pkey) & (idx_m > idx_p))
            if d < R:
                bitd = (i0 & d) != 0
            else:
                bitd = (i1 & (d // R)) != 0
            take = jnp.logical_xor(gt, bitd == asc)
            key = jnp.where(take, pkey, key)
            pay = jnp.where(take, ppay, pay)

    pixs = jnp.bitwise_and(lax.shift_right_logical(pay, 16), 0xFFFF)
    rmod = i0 % 16
    dup_later = jnp.zeros((R, C), dtype=jnp.bool_)
    for o in range(1, 16):
        eq = pixs == pltpu.roll(pixs, R - o, 0)
        dup_later = dup_later | (eq & (rmod < 16 - o))
    valid = jnp.logical_not(dup_later).astype(jnp.int32)
    mbit_e = mbit_ref[0].T
    w = jnp.bitwise_or(
        jnp.bitwise_or(jnp.left_shift(pay & 0xFFFF, 1), mbit_e),
        jnp.left_shift(valid, 17))
    pixs_ref[0] = pixs.T
    w_ref[0] = w.T


def _sort(skey, pix, mbit):
    pixs, w = pl.pallas_call(
        _sort_body,
        grid=(BS,),
        in_specs=[
            pl.BlockSpec((1, R, C), lambda b: (b, 0, 0)),
            pl.BlockSpec((1, R, C), lambda b: (b, 0, 0)),
            pl.BlockSpec((1, C, R), lambda b: (b, 0, 0)),
        ],
        out_specs=[pl.BlockSpec((1, C, R), lambda b: (b, 0, 0))] * 2,
        out_shape=[
            jax.ShapeDtypeStruct((BS, C, R), jnp.int32),
            jax.ShapeDtypeStruct((BS, C, R), jnp.int32),
        ],
    )(skey.reshape(BS, R, C), pix.reshape(BS, R, C), mbit.reshape(BS, C, R))
    return pixs.reshape(BS, N), w.reshape(BS, N)


_NC = 2
_NS = 16
_NW = _NC * _NS
_IMGS_PER_W = BS // _NW
_CH = 2048


def _scatter_tec_body(pixs_hbm, w_hbm, out_hbm, buf_v, pix_v, w_v):
    wid = lax.axis_index("s") * _NC + lax.axis_index("c")

    for li in range(_IMGS_PER_W):
        img = wid * _IMGS_PER_W + li
        base = img * N

        def init_body(i, carry):
            buf_v[pl.ds(i * 16, 16)] = jnp.full((16,), -1, jnp.int32)
            return carry

        lax.fori_loop(0, N // 16, init_body, 0)

        def chunk_body(c, carry):
            pltpu.sync_copy(pixs_hbm.at[pl.ds(base + c * _CH, _CH)], pix_v)
            pltpu.sync_copy(w_hbm.at[pl.ds(base + c * _CH, _CH)], w_v)

            def group_body(g, carry2):
                pixg = pix_v[pl.ds(g * 16, 16)]
                wg = w_v[pl.ds(g * 16, 16)]
                ok = jnp.bitwise_and(lax.shift_right_logical(wg, 17), 1) == 1
                plsc.store_scatter(buf_v, [pixg], wg, mask=ok)
                return carry2

            lax.fori_loop(0, _CH // 16, group_body, 0)
            return carry

        lax.fori_loop(0, N // _CH, chunk_body, 0)
        pltpu.sync_copy(buf_v, out_hbm.at[pl.ds(base, N)])


def _sc_scatter(pixs, wv):
    mesh = plsc.VectorSubcoreMesh(core_axis_name="c", subcore_axis_name="s")
    f = functools.partial(
        pl.kernel,
        mesh=mesh,
        compiler_params=pltpu.CompilerParams(needs_layout_passes=False),
        out_type=jax.ShapeDtypeStruct((BS * N,), jnp.int32),
        scratch_types=[
            pltpu.VMEM((N,), jnp.int32),
            pltpu.VMEM((_CH,), jnp.int32),
            pltpu.VMEM((_CH,), jnp.int32),
        ],
    )(_scatter_tec_body)
    return f(pixs.reshape(BS * N), wv.reshape(BS * N)).reshape(BS, N)


def kernel(depth, K, K_inv, RTinv_cam1, RT_cam2):
    RT = jnp.matmul(RT_cam2, RTinv_cam1)
    zneg, skey, pix, mbit = _project(depth, K, K_inv, RT)

    pixs, wv = _sort(skey, pix, mbit)
    winner = _sc_scatter(pixs, wv)

    valid = winner >= 0
    i = lax.shift_right_logical(winner, 1) & 0xFFFF
    mb = (winner & 1).astype(jnp.float32) * 4.0
    xg, yg = _grid_xy_np()
    g0 = jnp.take(jnp.asarray(xg).reshape(N), i)
    g1 = jnp.take(jnp.asarray(yg).reshape(N), i)
    v0 = jnp.where(valid, g0 + mb, -2.0)
    v1 = jnp.where(valid, -g1 + mb, -2.0)
    bil = jnp.stack([v0, v1], axis=1).reshape(BS, 2, W, W)
    return bil, zneg.reshape(BS, 1, W, W)
